# baseline (speedup 1.0000x reference)
import functools

import jax
import jax.numpy as jnp
from jax import lax
from jax.experimental import pallas as pl
from jax.experimental.pallas import tpu as pltpu
from jax.experimental.pallas import tpu_sc as plsc

N = 10000
E = 320000
D = 128

NC = 2
NS = 16
NW = NC * NS
E_PER_TILE = E // NW
CHUNK = 96
E_PAD = 10080
NCHUNK = E_PAD // CHUNK
SUP = 5
NCH_B = NCHUNK // SUP
NBUF = 3
ROWS_PER_TILE = 624
TAIL_ROWS = N - NS * ROWS_PER_TILE


def _sc_spmm(h, src, dst, w, zrows):
    mesh = plsc.VectorSubcoreMesh(core_axis_name="c", subcore_axis_name="s")

    @functools.partial(
        pl.kernel,
        out_type=jax.ShapeDtypeStruct((NC, N, D), jnp.float32),
        mesh=mesh,
        scratch_types=[
            pltpu.VMEM((NCH_B, CHUNK), jnp.int32),
            pltpu.VMEM((NCH_B, CHUNK), jnp.int32),
            pltpu.VMEM((NCH_B, CHUNK), jnp.float32),
            [pltpu.VMEM((CHUNK, D), jnp.float32)] * NBUF,
            [pltpu.SemaphoreType.DMA] * NBUF,
            [pltpu.SemaphoreType.DMA] * NBUF,
            pltpu.VMEM_SHARED((N, D), jnp.float32),
        ],
    )
    def spmm(h_hbm, src_hbm, dst_hbm, w_hbm, z_hbm, p_hbm,
             src_v, dst_v, w_v, rows, gsem, ssem, acc):
        cid = lax.axis_index("c")
        sid = lax.axis_index("s")
        wid = sid * NC + cid

        pltpu.sync_copy(z_hbm.at[pl.ds(0, ROWS_PER_TILE)],
                        acc.at[pl.ds(sid * ROWS_PER_TILE, ROWS_PER_TILE)])

        @pl.when(sid == NS - 1)
        def _():
            pltpu.sync_copy(z_hbm.at[pl.ds(0, TAIL_ROWS)],
                            acc.at[pl.ds(NS * ROWS_PER_TILE, TAIL_ROWS)])

        plsc.subcore_barrier()

        def issue_gather(k, b):
            pltpu.async_copy(h_hbm.at[src_v.at[k]], rows[b], gsem[b])

        def wait_gather(k, b):
            pltpu.make_async_copy(h_hbm.at[src_v.at[k]], rows[b], gsem[b]).wait()

        def issue_scatter(k, b):
            pltpu.async_copy(rows[b], acc.at[dst_v.at[k]], ssem[b], add=True)

        def wait_scatter(k, b):
            pltpu.make_async_copy(rows[b], acc.at[dst_v.at[k]], ssem[b]).wait()

        def multiply(k, b):
            def grp_body(gi, c2):
                w16 = w_v[k, pl.ds(gi * 16, 16)]
                for j in range(16):
                    ws = lax.broadcast(w16[j], (16,))
                    r = gi * 16 + j
                    for g in range(D // 16):
                        sl = (r, pl.ds(g * 16, 16))
                        rows[b][sl] = rows[b][sl] * ws
                return c2

            lax.fori_loop(0, CHUNK // 16, grp_body, 0)

        def step(k, b, first=False, last=False):
            b2 = (b + 2) % NBUF
            wait_gather(k, b)
            if not (first or last):
                wait_scatter(k - 1, b2)
            if not last:
                issue_gather(k + 2, b2)
            multiply(k, b)
            issue_scatter(k, b)

        def super_block(sb, carry):
            pltpu.sync_copy(src_hbm.at[wid, sb], src_v)
            pltpu.sync_copy(dst_hbm.at[wid, sb], dst_v)
            pltpu.sync_copy(w_hbm.at[wid, sb], w_v)
            issue_gather(0, 0)
            issue_gather(1, 1)
            step(0, 0, first=True)
            step(1, 1)
            step(2, 2)

            def triple(t, c2):
                for b in range(NBUF):
                    step(t * NBUF + b, b)
                return c2

            lax.fori_loop(1, NCH_B // NBUF - 1, triple, 0)
            step(NCH_B - 3, 0)
            step(NCH_B - 2, 1, last=True)
            step(NCH_B - 1, 2, last=True)
            for b in range(NBUF):
                wait_scatter(NCH_B - NBUF + b, b)
            return carry

        lax.fori_loop(0, SUP, super_block, 0)

        plsc.subcore_barrier()
        rsl = pl.ds(sid * ROWS_PER_TILE, ROWS_PER_TILE)
        pltpu.sync_copy(acc.at[rsl], p_hbm.at[cid, rsl])

        @pl.when(sid == NS - 1)
        def _():
            tsl = pl.ds(NS * ROWS_PER_TILE, TAIL_ROWS)
            pltpu.sync_copy(acc.at[tsl], p_hbm.at[cid, tsl])

    return spmm(h, src, dst, w, zrows)


BR = 1000


def _transform(h, w_ref, prm_ref):
    f = lax.dot_general(h, w_ref[...], (((1,), (1,)), ((), ())),
                        preferred_element_type=jnp.float32)
    f = jnp.maximum(f + prm_ref[0], 0.0)
    mean = jnp.mean(f, axis=1, keepdims=True)
    c = f - mean
    var = jnp.mean(c * c, axis=1, keepdims=True) + 1e-9
    return c * prm_ref[1] * lax.rsqrt(var) + prm_ref[2]


def _tc_stage1(x, pa, W0, P0):
    def body(x_ref, pa_ref, w_ref, prm_ref, h1_ref, f0_ref):
        h1_ref[...] = pa_ref[0] + pa_ref[1]
        f0_ref[...] = _transform(x_ref[...], w_ref, prm_ref)

    return pl.pallas_call(
        body,
        grid=(N // BR,),
        in_specs=[
            pl.BlockSpec((BR, D), lambda i: (i, 0)),
            pl.BlockSpec((NC, BR, D), lambda i: (0, i, 0)),
            pl.BlockSpec((D, D), lambda i: (0, 0)),
            pl.BlockSpec((3, D), lambda i: (0, 0)),
        ],
        out_specs=[
            pl.BlockSpec((BR, D), lambda i: (i, 0)),
            pl.BlockSpec((BR, D), lambda i: (i, 0)),
        ],
        out_shape=[
            jax.ShapeDtypeStruct((N, D), jnp.float32),
            jax.ShapeDtypeStruct((N, D), jnp.float32),
        ],
    )(x, pa, W0, P0)


def _tc_stage2(pb, h1, f0, W1, W2, P1, P2):
    def body(pb_ref, h1_ref, f0_ref, w1_ref, w2_ref, p1_ref, p2_ref, out_ref):
        h2 = pb_ref[0] + pb_ref[1]
        out_ref[:, 0:D] = f0_ref[...]
        out_ref[:, D:2 * D] = _transform(h1_ref[...], w1_ref, p1_ref)
        out_ref[:, 2 * D:3 * D] = _transform(h2, w2_ref, p2_ref)

    return pl.pallas_call(
        body,
        grid=(N // BR,),
        in_specs=[
            pl.BlockSpec((NC, BR, D), lambda i: (0, i, 0)),
            pl.BlockSpec((BR, D), lambda i: (i, 0)),
            pl.BlockSpec((BR, D), lambda i: (i, 0)),
            pl.BlockSpec((D, D), lambda i: (0, 0)),
            pl.BlockSpec((D, D), lambda i: (0, 0)),
            pl.BlockSpec((3, D), lambda i: (0, 0)),
            pl.BlockSpec((3, D), lambda i: (0, 0)),
        ],
        out_specs=pl.BlockSpec((BR, 3 * D), lambda i: (i, 0)),
        out_shape=jax.ShapeDtypeStruct((N, 3 * D), jnp.float32),
    )(pb, h1, f0, W1, W2, P1, P2)


def kernel(x, edge_index, edge_weight, W0, W1, W2, b0, b1, b2,
           s0, s1, s2, o0, o1, o2):
    pad = E_PAD - E_PER_TILE
    dst = jnp.pad(edge_index[0].reshape(NW, E_PER_TILE), ((0, 0), (0, pad))
                  ).reshape(NW, SUP, NCH_B, CHUNK)
    src = jnp.pad(edge_index[1].reshape(NW, E_PER_TILE), ((0, 0), (0, pad))
                  ).reshape(NW, SUP, NCH_B, CHUNK)
    w = jnp.pad(edge_weight.reshape(NW, E_PER_TILE), ((0, 0), (0, pad))
                ).reshape(NW, SUP, NCH_B, CHUNK)
    zrows = jnp.zeros((ROWS_PER_TILE, D), jnp.float32)
    P0 = jnp.stack([b0, s0, o0])
    P1 = jnp.stack([b1, s1, o1])
    P2 = jnp.stack([b2, s2, o2])

    pa = _sc_spmm(x, src, dst, w, zrows)
    h1, f0 = _tc_stage1(x, pa, W0, P0)
    pb = _sc_spmm(h1, src, dst, w, zrows)
    return _tc_stage2(pb, h1, f0, W1, W2, P1, P2)


# speedup vs baseline: 1.0140x; 1.0140x over previous
import functools

import jax
import jax.numpy as jnp
from jax import lax
from jax.experimental import pallas as pl
from jax.experimental.pallas import tpu as pltpu
from jax.experimental.pallas import tpu_sc as plsc

N = 10000
E = 320000
D = 128

NC = 2
NS = 16
NW = NC * NS
E_PER_TILE = E // NW
CHUNK = 80
E_PAD = 10080
NCHUNK = E_PAD // CHUNK
SUP = 3
NCH_B = NCHUNK // SUP
NBUF = 3
ROWS_PER_TILE = 624
TAIL_ROWS = N - NS * ROWS_PER_TILE


def _sc_spmm(h, src, dst, w, zrows):
    mesh = plsc.VectorSubcoreMesh(core_axis_name="c", subcore_axis_name="s")

    @functools.partial(
        pl.kernel,
        out_type=jax.ShapeDtypeStruct((NC, N, D), jnp.float32),
        mesh=mesh,
        scratch_types=[
            pltpu.VMEM((NCH_B, CHUNK), jnp.int32),
            pltpu.VMEM((NCH_B, CHUNK), jnp.int32),
            pltpu.VMEM((NCH_B, CHUNK), jnp.float32),
            [pltpu.VMEM((CHUNK, D), jnp.float32)] * NBUF,
            [pltpu.SemaphoreType.DMA] * NBUF,
            [pltpu.SemaphoreType.DMA] * NBUF,
            pltpu.VMEM_SHARED((N, D), jnp.float32),
        ],
    )
    def spmm(h_hbm, src_hbm, dst_hbm, w_hbm, z_hbm, p_hbm,
             src_v, dst_v, w_v, rows, gsem, ssem, acc):
        cid = lax.axis_index("c")
        sid = lax.axis_index("s")
        wid = sid * NC + cid

        pltpu.sync_copy(z_hbm.at[pl.ds(0, ROWS_PER_TILE)],
                        acc.at[pl.ds(sid * ROWS_PER_TILE, ROWS_PER_TILE)])

        @pl.when(sid == NS - 1)
        def _():
            pltpu.sync_copy(z_hbm.at[pl.ds(0, TAIL_ROWS)],
                            acc.at[pl.ds(NS * ROWS_PER_TILE, TAIL_ROWS)])

        plsc.subcore_barrier()

        def issue_gather(k, b):
            pltpu.async_copy(h_hbm.at[src_v.at[k]], rows[b], gsem[b])

        def wait_gather(k, b):
            pltpu.make_async_copy(h_hbm.at[src_v.at[k]], rows[b], gsem[b]).wait()

        def issue_scatter(k, b):
            pltpu.async_copy(rows[b], acc.at[dst_v.at[k]], ssem[b], add=True)

        def wait_scatter(k, b):
            pltpu.make_async_copy(rows[b], acc.at[dst_v.at[k]], ssem[b]).wait()

        def multiply(k, b):
            def grp_body(gi, c2):
                w16 = w_v[k, pl.ds(gi * 16, 16)]
                for j in range(16):
                    ws = lax.broadcast(w16[j], (16,))
                    r = gi * 16 + j
                    for g in range(D // 16):
                        sl = (r, pl.ds(g * 16, 16))
                        rows[b][sl] = rows[b][sl] * ws
                return c2

            lax.fori_loop(0, CHUNK // 16, grp_body, 0)

        def step(k, b, first=False, last=False):
            b2 = (b + 2) % NBUF
            wait_gather(k, b)
            if not (first or last):
                wait_scatter(k - 1, b2)
            if not last:
                issue_gather(k + 2, b2)
            multiply(k, b)
            issue_scatter(k, b)

        def super_block(sb, carry):
            pltpu.sync_copy(src_hbm.at[wid, sb], src_v)
            pltpu.sync_copy(dst_hbm.at[wid, sb], dst_v)
            pltpu.sync_copy(w_hbm.at[wid, sb], w_v)
            issue_gather(0, 0)
            issue_gather(1, 1)
            step(0, 0, first=True)
            step(1, 1)
            step(2, 2)

            def triple(t, c2):
                for b in range(NBUF):
                    step(t * NBUF + b, b)
                return c2

            lax.fori_loop(1, NCH_B // NBUF - 1, triple, 0)
            step(NCH_B - 3, 0)
            step(NCH_B - 2, 1, last=True)
            step(NCH_B - 1, 2, last=True)
            for b in range(NBUF):
                wait_scatter(NCH_B - NBUF + b, b)
            return carry

        lax.fori_loop(0, SUP, super_block, 0)

        plsc.subcore_barrier()
        rsl = pl.ds(sid * ROWS_PER_TILE, ROWS_PER_TILE)
        pltpu.sync_copy(acc.at[rsl], p_hbm.at[cid, rsl])

        @pl.when(sid == NS - 1)
        def _():
            tsl = pl.ds(NS * ROWS_PER_TILE, TAIL_ROWS)
            pltpu.sync_copy(acc.at[tsl], p_hbm.at[cid, tsl])

    return spmm(h, src, dst, w, zrows)


BR = 1000


def _transform(h, w_ref, prm_ref):
    f = lax.dot_general(h, w_ref[...], (((1,), (1,)), ((), ())),
                        preferred_element_type=jnp.float32)
    f = jnp.maximum(f + prm_ref[0], 0.0)
    mean = jnp.mean(f, axis=1, keepdims=True)
    c = f - mean
    var = jnp.mean(c * c, axis=1, keepdims=True) + 1e-9
    return c * prm_ref[1] * lax.rsqrt(var) + prm_ref[2]


def _tc_stage1(x, pa, W0, P0):
    def body(x_ref, pa_ref, w_ref, prm_ref, h1_ref, f0_ref):
        h1_ref[...] = pa_ref[0] + pa_ref[1]
        f0_ref[...] = _transform(x_ref[...], w_ref, prm_ref)

    return pl.pallas_call(
        body,
        grid=(N // BR,),
        in_specs=[
            pl.BlockSpec((BR, D), lambda i: (i, 0)),
            pl.BlockSpec((NC, BR, D), lambda i: (0, i, 0)),
            pl.BlockSpec((D, D), lambda i: (0, 0)),
            pl.BlockSpec((3, D), lambda i: (0, 0)),
        ],
        out_specs=[
            pl.BlockSpec((BR, D), lambda i: (i, 0)),
            pl.BlockSpec((BR, D), lambda i: (i, 0)),
        ],
        out_shape=[
            jax.ShapeDtypeStruct((N, D), jnp.float32),
            jax.ShapeDtypeStruct((N, D), jnp.float32),
        ],
    )(x, pa, W0, P0)


def _tc_stage2(pb, h1, f0, W1, W2, P1, P2):
    def body(pb_ref, h1_ref, f0_ref, w1_ref, w2_ref, p1_ref, p2_ref, out_ref):
        h2 = pb_ref[0] + pb_ref[1]
        out_ref[:, 0:D] = f0_ref[...]
        out_ref[:, D:2 * D] = _transform(h1_ref[...], w1_ref, p1_ref)
        out_ref[:, 2 * D:3 * D] = _transform(h2, w2_ref, p2_ref)

    return pl.pallas_call(
        body,
        grid=(N // BR,),
        in_specs=[
            pl.BlockSpec((NC, BR, D), lambda i: (0, i, 0)),
            pl.BlockSpec((BR, D), lambda i: (i, 0)),
            pl.BlockSpec((BR, D), lambda i: (i, 0)),
            pl.BlockSpec((D, D), lambda i: (0, 0)),
            pl.BlockSpec((D, D), lambda i: (0, 0)),
            pl.BlockSpec((3, D), lambda i: (0, 0)),
            pl.BlockSpec((3, D), lambda i: (0, 0)),
        ],
        out_specs=pl.BlockSpec((BR, 3 * D), lambda i: (i, 0)),
        out_shape=jax.ShapeDtypeStruct((N, 3 * D), jnp.float32),
    )(pb, h1, f0, W1, W2, P1, P2)


def kernel(x, edge_index, edge_weight, W0, W1, W2, b0, b1, b2,
           s0, s1, s2, o0, o1, o2):
    pad = E_PAD - E_PER_TILE
    dst = jnp.pad(edge_index[0].reshape(NW, E_PER_TILE), ((0, 0), (0, pad))
                  ).reshape(NW, SUP, NCH_B, CHUNK)
    src = jnp.pad(edge_index[1].reshape(NW, E_PER_TILE), ((0, 0), (0, pad))
                  ).reshape(NW, SUP, NCH_B, CHUNK)
    w = jnp.pad(edge_weight.reshape(NW, E_PER_TILE), ((0, 0), (0, pad))
                ).reshape(NW, SUP, NCH_B, CHUNK)
    zrows = jnp.zeros((ROWS_PER_TILE, D), jnp.float32)
    P0 = jnp.stack([b0, s0, o0])
    P1 = jnp.stack([b1, s1, o1])
    P2 = jnp.stack([b2, s2, o2])

    pa = _sc_spmm(x, src, dst, w, zrows)
    h1, f0 = _tc_stage1(x, pa, W0, P0)
    pb = _sc_spmm(h1, src, dst, w, zrows)
    return _tc_stage2(pb, h1, f0, W1, W2, P1, P2)


# speedup vs baseline: 1.0215x; 1.0075x over previous
import functools

import jax
import jax.numpy as jnp
from jax import lax
from jax.experimental import pallas as pl
from jax.experimental.pallas import tpu as pltpu
from jax.experimental.pallas import tpu_sc as plsc

N = 10000
E = 320000
D = 128

NC = 2
NS = 16
NW = NC * NS
E_PER_TILE = E // NW
CHUNK = 80
E_PAD = 10080
NCHUNK = E_PAD // CHUNK
SUP = 3
NCH_B = NCHUNK // SUP
NBUF = 3
ROWS_PER_TILE = 624
TAIL_ROWS = N - NS * ROWS_PER_TILE


def _sc_spmm(h, src, dst, w, zrows):
    mesh = plsc.VectorSubcoreMesh(core_axis_name="c", subcore_axis_name="s")

    @functools.partial(
        pl.kernel,
        out_type=jax.ShapeDtypeStruct((NC, N, D), jnp.float32),
        mesh=mesh,
        scratch_types=[
            pltpu.VMEM((NCH_B, CHUNK), jnp.int32),
            pltpu.VMEM((NCH_B, CHUNK), jnp.int32),
            pltpu.VMEM((NCH_B, CHUNK), jnp.float32),
            [pltpu.VMEM((CHUNK, D), jnp.float32)] * NBUF,
            [pltpu.SemaphoreType.DMA] * NBUF,
            [pltpu.SemaphoreType.DMA] * NBUF,
            pltpu.VMEM_SHARED((N, D), jnp.float32),
        ],
    )
    def spmm(h_hbm, src_hbm, dst_hbm, w_hbm, z_hbm, p_hbm,
             src_v, dst_v, w_v, rows, gsem, ssem, acc):
        cid = lax.axis_index("c")
        sid = lax.axis_index("s")
        wid = sid * NC + cid

        pltpu.sync_copy(z_hbm.at[pl.ds(0, ROWS_PER_TILE)],
                        acc.at[pl.ds(sid * ROWS_PER_TILE, ROWS_PER_TILE)])

        @pl.when(sid == NS - 1)
        def _():
            pltpu.sync_copy(z_hbm.at[pl.ds(0, TAIL_ROWS)],
                            acc.at[pl.ds(NS * ROWS_PER_TILE, TAIL_ROWS)])

        plsc.subcore_barrier()

        def issue_gather(k, b):
            pltpu.async_copy(h_hbm.at[src_v.at[k]], rows[b], gsem[b])

        def wait_gather(k, b):
            pltpu.make_async_copy(h_hbm.at[src_v.at[k]], rows[b], gsem[b]).wait()

        def issue_scatter(k, b):
            pltpu.async_copy(rows[b], acc.at[dst_v.at[k]], ssem[b], add=True)

        def wait_scatter(k, b):
            pltpu.make_async_copy(rows[b], acc.at[dst_v.at[k]], ssem[b]).wait()

        def multiply(k, b):
            def grp_body(gi, c2):
                w16 = w_v[k, pl.ds(gi * 16, 16)]
                for j in range(16):
                    ws = lax.broadcast(w16[j], (16,))
                    r = gi * 16 + j
                    for g in range(D // 16):
                        sl = (r, pl.ds(g * 16, 16))
                        rows[b][sl] = rows[b][sl] * ws
                return c2

            lax.fori_loop(0, CHUNK // 16, grp_body, 0)

        def step(k, b, first=False, last=False):
            b2 = (b + 2) % NBUF
            wait_gather(k, b)
            if not (first or last):
                wait_scatter(k - 1, b2)
            if not last:
                issue_gather(k + 2, b2)
            multiply(k, b)
            issue_scatter(k, b)

        def super_block(sb, carry):
            pltpu.sync_copy(src_hbm.at[wid, sb], src_v)
            pltpu.sync_copy(dst_hbm.at[wid, sb], dst_v)
            pltpu.sync_copy(w_hbm.at[wid, sb], w_v)
            issue_gather(0, 0)
            issue_gather(1, 1)
            step(0, 0, first=True)
            step(1, 1)
            step(2, 2)

            def triple(t, c2):
                for b in range(NBUF):
                    step(t * NBUF + b, b)
                return c2

            lax.fori_loop(1, NCH_B // NBUF - 1, triple, 0)
            step(NCH_B - 3, 0)
            step(NCH_B - 2, 1, last=True)
            step(NCH_B - 1, 2, last=True)
            for b in range(NBUF):
                wait_scatter(NCH_B - NBUF + b, b)
            return carry

        lax.fori_loop(0, SUP, super_block, 0)

        plsc.subcore_barrier()
        rsl = pl.ds(sid * ROWS_PER_TILE, ROWS_PER_TILE)
        pltpu.sync_copy(acc.at[rsl], p_hbm.at[cid, rsl])

        @pl.when(sid == NS - 1)
        def _():
            tsl = pl.ds(NS * ROWS_PER_TILE, TAIL_ROWS)
            pltpu.sync_copy(acc.at[tsl], p_hbm.at[cid, tsl])

    return spmm(h, src, dst, w, zrows)


BR = 1000


def _transform(h, w_ref, prm_ref):
    f = lax.dot_general(h, w_ref[...], (((1,), (1,)), ((), ())),
                        preferred_element_type=jnp.float32)
    f = jnp.maximum(f + prm_ref[0], 0.0)
    mean = jnp.mean(f, axis=1, keepdims=True)
    c = f - mean
    var = jnp.mean(c * c, axis=1, keepdims=True) + 1e-9
    return c * prm_ref[1] * lax.rsqrt(var) + prm_ref[2]


def _tc_transform(h, W, P):
    def body(h_ref, w_ref, prm_ref, f_ref):
        f_ref[...] = _transform(h_ref[...], w_ref, prm_ref)

    return pl.pallas_call(
        body,
        grid=(N // BR,),
        in_specs=[
            pl.BlockSpec((BR, D), lambda i: (i, 0)),
            pl.BlockSpec((D, D), lambda i: (0, 0)),
            pl.BlockSpec((3, D), lambda i: (0, 0)),
        ],
        out_specs=pl.BlockSpec((BR, D), lambda i: (i, 0)),
        out_shape=jax.ShapeDtypeStruct((N, D), jnp.float32),
    )(h, W, P)


def _tc_combine(pa):
    def body(pa_ref, h_ref):
        h_ref[...] = pa_ref[0] + pa_ref[1]

    return pl.pallas_call(
        body,
        grid=(N // BR,),
        in_specs=[pl.BlockSpec((NC, BR, D), lambda i: (0, i, 0))],
        out_specs=pl.BlockSpec((BR, D), lambda i: (i, 0)),
        out_shape=jax.ShapeDtypeStruct((N, D), jnp.float32),
    )(pa)


def _tc_final(pb, f0, f1, W2, P2):
    def body(pb_ref, f0_ref, f1_ref, w2_ref, p2_ref, out_ref):
        h2 = pb_ref[0] + pb_ref[1]
        out_ref[:, 0:D] = f0_ref[...]
        out_ref[:, D:2 * D] = f1_ref[...]
        out_ref[:, 2 * D:3 * D] = _transform(h2, w2_ref, p2_ref)

    return pl.pallas_call(
        body,
        grid=(N // BR,),
        in_specs=[
            pl.BlockSpec((NC, BR, D), lambda i: (0, i, 0)),
            pl.BlockSpec((BR, D), lambda i: (i, 0)),
            pl.BlockSpec((BR, D), lambda i: (i, 0)),
            pl.BlockSpec((D, D), lambda i: (0, 0)),
            pl.BlockSpec((3, D), lambda i: (0, 0)),
        ],
        out_specs=pl.BlockSpec((BR, 3 * D), lambda i: (i, 0)),
        out_shape=jax.ShapeDtypeStruct((N, 3 * D), jnp.float32),
    )(pb, f0, f1, W2, P2)


def kernel(x, edge_index, edge_weight, W0, W1, W2, b0, b1, b2,
           s0, s1, s2, o0, o1, o2):
    pad = E_PAD - E_PER_TILE
    dst = jnp.pad(edge_index[0].reshape(NW, E_PER_TILE), ((0, 0), (0, pad))
                  ).reshape(NW, SUP, NCH_B, CHUNK)
    src = jnp.pad(edge_index[1].reshape(NW, E_PER_TILE), ((0, 0), (0, pad))
                  ).reshape(NW, SUP, NCH_B, CHUNK)
    w = jnp.pad(edge_weight.reshape(NW, E_PER_TILE), ((0, 0), (0, pad))
                ).reshape(NW, SUP, NCH_B, CHUNK)
    zrows = jnp.zeros((ROWS_PER_TILE, D), jnp.float32)
    P0 = jnp.stack([b0, s0, o0])
    P1 = jnp.stack([b1, s1, o1])
    P2 = jnp.stack([b2, s2, o2])

    pa = _sc_spmm(x, src, dst, w, zrows)
    f0 = _tc_transform(x, W0, P0)
    h1 = _tc_combine(pa)
    pb = _sc_spmm(h1, src, dst, w, zrows)
    f1 = _tc_transform(h1, W1, P1)
    return _tc_final(pb, f0, f1, W2, P2)


# speedup vs baseline: 1.0237x; 1.0021x over previous
import functools

import jax
import jax.numpy as jnp
from jax import lax
from jax.experimental import pallas as pl
from jax.experimental.pallas import tpu as pltpu
from jax.experimental.pallas import tpu_sc as plsc

N = 10000
E = 320000
D = 128

NC = 2
NS = 16
NW = NC * NS
E_PER_TILE = E // NW
CHUNK = 80
E_PAD = 10080
NCHUNK = E_PAD // CHUNK
SUP = 3
NCH_B = NCHUNK // SUP
NBUF = 3
ROWS_PER_TILE = 624
TAIL_ROWS = N - NS * ROWS_PER_TILE


def _sc_spmm(h, src, dst, w, zrows):
    mesh = plsc.VectorSubcoreMesh(core_axis_name="c", subcore_axis_name="s")

    @functools.partial(
        pl.kernel,
        out_type=jax.ShapeDtypeStruct((NC, N, D), jnp.float32),
        mesh=mesh,
        scratch_types=[
            pltpu.VMEM((NCH_B, CHUNK), jnp.int32),
            pltpu.VMEM((NCH_B, CHUNK), jnp.int32),
            pltpu.VMEM((NCH_B, CHUNK), jnp.float32),
            [pltpu.VMEM((CHUNK, D), jnp.float32)] * NBUF,
            [pltpu.SemaphoreType.DMA] * NBUF,
            [pltpu.SemaphoreType.DMA] * NBUF,
            pltpu.VMEM_SHARED((N, D), jnp.float32),
        ],
    )
    def spmm(h_hbm, src_hbm, dst_hbm, w_hbm, z_hbm, p_hbm,
             src_v, dst_v, w_v, rows, gsem, ssem, acc):
        cid = lax.axis_index("c")
        sid = lax.axis_index("s")
        wid = sid * NC + cid

        def issue_gather(k, b):
            pltpu.async_copy(h_hbm.at[src_v.at[k]], rows[b], gsem[b])

        def wait_gather(k, b):
            pltpu.make_async_copy(h_hbm.at[src_v.at[k]], rows[b], gsem[b]).wait()

        def issue_scatter(k, b):
            pltpu.async_copy(rows[b], acc.at[dst_v.at[k]], ssem[b], add=True)

        def wait_scatter(k, b):
            pltpu.make_async_copy(rows[b], acc.at[dst_v.at[k]], ssem[b]).wait()

        def multiply(k, b):
            def grp_body(gi, c2):
                w16 = w_v[k, pl.ds(gi * 16, 16)]
                for j in range(16):
                    ws = lax.broadcast(w16[j], (16,))
                    r = gi * 16 + j
                    for g in range(D // 16):
                        sl = (r, pl.ds(g * 16, 16))
                        rows[b][sl] = rows[b][sl] * ws
                return c2

            lax.fori_loop(0, CHUNK // 16, grp_body, 0)

        def step(k, b, first=False, last=False):
            b2 = (b + 2) % NBUF
            wait_gather(k, b)
            if not (first or last):
                wait_scatter(k - 1, b2)
            if not last:
                issue_gather(k + 2, b2)
            multiply(k, b)
            issue_scatter(k, b)

        def super_block(sb, carry):
            pltpu.sync_copy(src_hbm.at[wid, sb], src_v)
            pltpu.sync_copy(dst_hbm.at[wid, sb], dst_v)
            pltpu.sync_copy(w_hbm.at[wid, sb], w_v)
            issue_gather(0, 0)
            issue_gather(1, 1)

            @pl.when(sb == 0)
            def _():
                pltpu.sync_copy(
                    z_hbm.at[pl.ds(0, ROWS_PER_TILE)],
                    acc.at[pl.ds(sid * ROWS_PER_TILE, ROWS_PER_TILE)])

                @pl.when(sid == NS - 1)
                def _():
                    pltpu.sync_copy(
                        z_hbm.at[pl.ds(0, TAIL_ROWS)],
                        acc.at[pl.ds(NS * ROWS_PER_TILE, TAIL_ROWS)])

                plsc.subcore_barrier()

            step(0, 0, first=True)
            step(1, 1)
            step(2, 2)

            def triple(t, c2):
                for b in range(NBUF):
                    step(t * NBUF + b, b)
                return c2

            lax.fori_loop(1, NCH_B // NBUF - 1, triple, 0)
            step(NCH_B - 3, 0)
            step(NCH_B - 2, 1, last=True)
            step(NCH_B - 1, 2, last=True)
            for b in range(NBUF):
                wait_scatter(NCH_B - NBUF + b, b)
            return carry

        lax.fori_loop(0, SUP, super_block, 0)

        plsc.subcore_barrier()
        rsl = pl.ds(sid * ROWS_PER_TILE, ROWS_PER_TILE)
        pltpu.sync_copy(acc.at[rsl], p_hbm.at[cid, rsl])

        @pl.when(sid == NS - 1)
        def _():
            tsl = pl.ds(NS * ROWS_PER_TILE, TAIL_ROWS)
            pltpu.sync_copy(acc.at[tsl], p_hbm.at[cid, tsl])

    return spmm(h, src, dst, w, zrows)


BR = 1000


def _transform(h, w_ref, prm_ref):
    f = lax.dot_general(h, w_ref[...], (((1,), (1,)), ((), ())),
                        preferred_element_type=jnp.float32)
    f = jnp.maximum(f + prm_ref[0], 0.0)
    mean = jnp.mean(f, axis=1, keepdims=True)
    c = f - mean
    var = jnp.mean(c * c, axis=1, keepdims=True) + 1e-9
    return c * prm_ref[1] * lax.rsqrt(var) + prm_ref[2]


def _tc_transform(h, W, P):
    def body(h_ref, w_ref, prm_ref, f_ref):
        f_ref[...] = _transform(h_ref[...], w_ref, prm_ref)

    return pl.pallas_call(
        body,
        grid=(N // BR,),
        in_specs=[
            pl.BlockSpec((BR, D), lambda i: (i, 0)),
            pl.BlockSpec((D, D), lambda i: (0, 0)),
            pl.BlockSpec((3, D), lambda i: (0, 0)),
        ],
        out_specs=pl.BlockSpec((BR, D), lambda i: (i, 0)),
        out_shape=jax.ShapeDtypeStruct((N, D), jnp.float32),
    )(h, W, P)


def _tc_combine(pa):
    def body(pa_ref, h_ref):
        h_ref[...] = pa_ref[0] + pa_ref[1]

    return pl.pallas_call(
        body,
        grid=(N // BR,),
        in_specs=[pl.BlockSpec((NC, BR, D), lambda i: (0, i, 0))],
        out_specs=pl.BlockSpec((BR, D), lambda i: (i, 0)),
        out_shape=jax.ShapeDtypeStruct((N, D), jnp.float32),
    )(pa)


def _tc_final(pb, f0, f1, W2, P2):
    def body(pb_ref, f0_ref, f1_ref, w2_ref, p2_ref, out_ref):
        h2 = pb_ref[0] + pb_ref[1]
        out_ref[:, 0:D] = f0_ref[...]
        out_ref[:, D:2 * D] = f1_ref[...]
        out_ref[:, 2 * D:3 * D] = _transform(h2, w2_ref, p2_ref)

    return pl.pallas_call(
        body,
        grid=(N // BR,),
        in_specs=[
            pl.BlockSpec((NC, BR, D), lambda i: (0, i, 0)),
            pl.BlockSpec((BR, D), lambda i: (i, 0)),
            pl.BlockSpec((BR, D), lambda i: (i, 0)),
            pl.BlockSpec((D, D), lambda i: (0, 0)),
            pl.BlockSpec((3, D), lambda i: (0, 0)),
        ],
        out_specs=pl.BlockSpec((BR, 3 * D), lambda i: (i, 0)),
        out_shape=jax.ShapeDtypeStruct((N, 3 * D), jnp.float32),
    )(pb, f0, f1, W2, P2)


def kernel(x, edge_index, edge_weight, W0, W1, W2, b0, b1, b2,
           s0, s1, s2, o0, o1, o2):
    pad = E_PAD - E_PER_TILE
    dst = jnp.pad(edge_index[0].reshape(NW, E_PER_TILE), ((0, 0), (0, pad))
                  ).reshape(NW, SUP, NCH_B, CHUNK)
    src = jnp.pad(edge_index[1].reshape(NW, E_PER_TILE), ((0, 0), (0, pad))
                  ).reshape(NW, SUP, NCH_B, CHUNK)
    w = jnp.pad(edge_weight.reshape(NW, E_PER_TILE), ((0, 0), (0, pad))
                ).reshape(NW, SUP, NCH_B, CHUNK)
    zrows = jnp.zeros((ROWS_PER_TILE, D), jnp.float32)
    P0 = jnp.stack([b0, s0, o0])
    P1 = jnp.stack([b1, s1, o1])
    P2 = jnp.stack([b2, s2, o2])

    pa = _sc_spmm(x, src, dst, w, zrows)
    f0 = _tc_transform(x, W0, P0)
    h1 = _tc_combine(pa)
    pb = _sc_spmm(h1, src, dst, w, zrows)
    f1 = _tc_transform(h1, W1, P1)
    return _tc_final(pb, f0, f1, W2, P2)
